# baseline (device time: 161120 ns/iter reference)
import jax
import jax.numpy as jnp
from jax import lax
from jax.experimental import pallas as pl
from jax.experimental.pallas import tpu as pltpu

N_DEV = 32
B = 2
SQ = 256
D_MODEL = 768
H_LOC = 8
GQA_GROUP = 4
KV_LOC = H_LOC // GQA_GROUP
DH = 64
SKV = 512
ROWS = B * SQ
N_CHUNKS = N_DEV
CHUNK_ROWS = ROWS // N_CHUNKS
CHUNKS_PER_B = SQ // CHUNK_ROWS


def _body(x_ref, wq_ref, wo_ref, k_ref, v_ref, out_ref,
          acc_ref, recv_ref, send_sem, rs_recv_sems, ag_recv_sems):
    my = lax.axis_index("i")
    left = (my + N_DEV - 1) % N_DEV
    right = (my + 1) % N_DEV

    barrier = pltpu.get_barrier_semaphore()
    for nbr in (left, right):
        pl.semaphore_signal(barrier, inc=1, device_id=(nbr,),
                            device_id_type=pl.DeviceIdType.MESH)
    pl.semaphore_wait(barrier, 2)

    for b in range(B):
        qb = jnp.dot(x_ref[b], wq_ref[...],
                     preferred_element_type=jnp.float32)
        outs = []
        for h in range(H_LOC):
            q = qb[:, h * DH:(h + 1) * DH]
            k = k_ref[b, h // GQA_GROUP]
            v = v_ref[b, h // GQA_GROUP]
            s = jnp.dot(q, k.T, preferred_element_type=jnp.float32) * 0.125
            m = jnp.max(s, axis=1, keepdims=True)
            p = jnp.exp(s - m)
            l = jnp.sum(p, axis=1, keepdims=True)
            outs.append(jnp.dot(p, v, preferred_element_type=jnp.float32) / l)
        o = jnp.concatenate(outs, axis=1)
        pb = jnp.dot(o, wo_ref[...],
                     preferred_element_type=jnp.float32)
        acc_ref[pl.ds(b * CHUNKS_PER_B, CHUNKS_PER_B)] = pb.reshape(
            CHUNKS_PER_B, CHUNK_ROWS, D_MODEL)

    def rs_step(t, carry):
        cs = (my + N_DEV - t) % N_DEV
        cr = (my + N_DEV - t - 1) % N_DEV
        send = pltpu.make_async_remote_copy(
            src_ref=acc_ref.at[cs],
            dst_ref=recv_ref.at[cs],
            send_sem=send_sem,
            recv_sem=rs_recv_sems.at[cs],
            device_id=(right,),
            device_id_type=pl.DeviceIdType.MESH,
        )
        send.start()
        recv = pltpu.make_async_remote_copy(
            src_ref=acc_ref.at[cr],
            dst_ref=recv_ref.at[cr],
            send_sem=send_sem,
            recv_sem=rs_recv_sems.at[cr],
            device_id=(right,),
            device_id_type=pl.DeviceIdType.MESH,
        )
        recv.wait_recv()
        acc_ref[pl.ds(cr, 1)] = acc_ref[pl.ds(cr, 1)] + recv_ref[pl.ds(cr, 1)]
        send.wait_send()
        return carry

    lax.fori_loop(0, N_DEV - 1, rs_step, 0)

    def ag_step(t, carry):
        cs = (my + N_DEV + 1 - t) % N_DEV
        cr = (my + N_DEV - t) % N_DEV
        send = pltpu.make_async_remote_copy(
            src_ref=acc_ref.at[cs],
            dst_ref=acc_ref.at[cs],
            send_sem=send_sem,
            recv_sem=ag_recv_sems.at[cs],
            device_id=(right,),
            device_id_type=pl.DeviceIdType.MESH,
        )
        send.start()
        recv = pltpu.make_async_remote_copy(
            src_ref=acc_ref.at[cr],
            dst_ref=acc_ref.at[cr],
            send_sem=send_sem,
            recv_sem=ag_recv_sems.at[cr],
            device_id=(right,),
            device_id_type=pl.DeviceIdType.MESH,
        )
        recv.wait_recv()
        send.wait_send()
        return carry

    lax.fori_loop(0, N_DEV - 1, ag_step, 0)

    for b in range(B):
        out_ref[b] = acc_ref[b * CHUNKS_PER_B:(b + 1) * CHUNKS_PER_B].reshape(
            SQ, D_MODEL)


def kernel(x, Wq, Wo, K_ext, V_ext):
    i = lax.axis_index("i")
    k_loc = lax.dynamic_slice_in_dim(K_ext, KV_LOC * i, KV_LOC, axis=2)
    v_loc = lax.dynamic_slice_in_dim(V_ext, KV_LOC * i, KV_LOC, axis=2)
    k_loc = k_loc.transpose(0, 2, 1, 3)
    v_loc = v_loc.transpose(0, 2, 1, 3)

    return pl.pallas_call(
        _body,
        out_shape=jax.ShapeDtypeStruct((B, SQ, D_MODEL), jnp.float32),
        in_specs=[pl.BlockSpec(memory_space=pltpu.VMEM)] * 5,
        out_specs=pl.BlockSpec(memory_space=pltpu.VMEM),
        scratch_shapes=[
            pltpu.VMEM((N_CHUNKS, CHUNK_ROWS, D_MODEL), jnp.float32),
            pltpu.VMEM((N_CHUNKS, CHUNK_ROWS, D_MODEL), jnp.float32),
            pltpu.SemaphoreType.DMA,
            pltpu.SemaphoreType.DMA((N_CHUNKS,)),
            pltpu.SemaphoreType.DMA((N_CHUNKS,)),
        ],
        compiler_params=pltpu.CompilerParams(collective_id=0),
    )(x, Wq, Wo, k_loc, v_loc)


# device time: 68483 ns/iter; 2.3527x vs baseline; 2.3527x over previous
import jax
import jax.numpy as jnp
from jax import lax
from jax.experimental import pallas as pl
from jax.experimental.pallas import tpu as pltpu

N_DEV = 32
LOG2_N = 5
B = 2
SQ = 256
D_MODEL = 768
H_LOC = 8
GQA_GROUP = 4
KV_LOC = H_LOC // GQA_GROUP
DH = 64
SKV = 512
ROWS = B * SQ
N_CHUNKS = N_DEV
CHUNK_ROWS = ROWS // N_CHUNKS
CHUNKS_PER_B = SQ // CHUNK_ROWS


def _body(x_ref, wq_ref, wo_ref, k_ref, v_ref, out_ref,
          acc_ref, recv_ref, send_sem, rs_sems, ag_sems):
    my = lax.axis_index("i")

    barrier = pltpu.get_barrier_semaphore()
    for k in range(LOG2_N):
        pl.semaphore_signal(barrier, inc=1, device_id=(my ^ (1 << k),),
                            device_id_type=pl.DeviceIdType.MESH)
    pl.semaphore_wait(barrier, LOG2_N)

    for b in range(B):
        qb = jnp.dot(x_ref[b], wq_ref[...],
                     preferred_element_type=jnp.float32)
        outs = []
        for h in range(H_LOC):
            q = qb[:, h * DH:(h + 1) * DH]
            k = k_ref[b, h // GQA_GROUP]
            v = v_ref[b, h // GQA_GROUP]
            s = jnp.dot(q, k.T, preferred_element_type=jnp.float32) * 0.125
            m = jnp.max(s, axis=1, keepdims=True)
            p = jnp.exp(s - m)
            l = jnp.sum(p, axis=1, keepdims=True)
            outs.append(jnp.dot(p, v, preferred_element_type=jnp.float32) / l)
        o = jnp.concatenate(outs, axis=1)
        pb = jnp.dot(o, wo_ref[...],
                     preferred_element_type=jnp.float32)
        acc_ref[pl.ds(b * CHUNKS_PER_B, CHUNKS_PER_B)] = pb.reshape(
            CHUNKS_PER_B, CHUNK_ROWS, D_MODEL)

    cur_start = jnp.int32(0)
    for k in range(LOG2_N):
        half = 16 >> k
        bit = (my >> k) & 1
        keep_start = cur_start + bit * half
        send_start = cur_start + (1 - bit) * half
        partner = my ^ (1 << k)
        send = pltpu.make_async_remote_copy(
            src_ref=acc_ref.at[pl.ds(send_start, half)],
            dst_ref=recv_ref.at[pl.ds(send_start, half)],
            send_sem=send_sem,
            recv_sem=rs_sems.at[k],
            device_id=(partner,),
            device_id_type=pl.DeviceIdType.MESH,
        )
        send.start()
        recv = pltpu.make_async_remote_copy(
            src_ref=acc_ref.at[pl.ds(keep_start, half)],
            dst_ref=recv_ref.at[pl.ds(keep_start, half)],
            send_sem=send_sem,
            recv_sem=rs_sems.at[k],
            device_id=(partner,),
            device_id_type=pl.DeviceIdType.MESH,
        )
        recv.wait_recv()
        acc_ref[pl.ds(keep_start, half)] = (
            acc_ref[pl.ds(keep_start, half)]
            + recv_ref[pl.ds(keep_start, half)])
        send.wait_send()
        cur_start = keep_start

    own_start = jnp.int32(0)
    for k in range(LOG2_N):
        own_start = own_start + (((my >> k) & 1) << (LOG2_N - 1 - k))

    for j in range(LOG2_N):
        sz = 1 << j
        partner = my ^ (16 >> j)
        partner_start = own_start ^ sz
        send = pltpu.make_async_remote_copy(
            src_ref=acc_ref.at[pl.ds(own_start, sz)],
            dst_ref=acc_ref.at[pl.ds(own_start, sz)],
            send_sem=send_sem,
            recv_sem=ag_sems.at[j],
            device_id=(partner,),
            device_id_type=pl.DeviceIdType.MESH,
        )
        send.start()
        recv = pltpu.make_async_remote_copy(
            src_ref=acc_ref.at[pl.ds(partner_start, sz)],
            dst_ref=acc_ref.at[pl.ds(partner_start, sz)],
            send_sem=send_sem,
            recv_sem=ag_sems.at[j],
            device_id=(partner,),
            device_id_type=pl.DeviceIdType.MESH,
        )
        recv.wait_recv()
        send.wait_send()
        own_start = jnp.minimum(own_start, partner_start)

    for b in range(B):
        out_ref[b] = acc_ref[b * CHUNKS_PER_B:(b + 1) * CHUNKS_PER_B].reshape(
            SQ, D_MODEL)


def kernel(x, Wq, Wo, K_ext, V_ext):
    i = lax.axis_index("i")
    k_loc = lax.dynamic_slice_in_dim(K_ext, KV_LOC * i, KV_LOC, axis=2)
    v_loc = lax.dynamic_slice_in_dim(V_ext, KV_LOC * i, KV_LOC, axis=2)
    k_loc = k_loc.transpose(0, 2, 1, 3)
    v_loc = v_loc.transpose(0, 2, 1, 3)

    return pl.pallas_call(
        _body,
        out_shape=jax.ShapeDtypeStruct((B, SQ, D_MODEL), jnp.float32),
        in_specs=[pl.BlockSpec(memory_space=pltpu.VMEM)] * 5,
        out_specs=pl.BlockSpec(memory_space=pltpu.VMEM),
        scratch_shapes=[
            pltpu.VMEM((N_CHUNKS, CHUNK_ROWS, D_MODEL), jnp.float32),
            pltpu.VMEM((N_CHUNKS, CHUNK_ROWS, D_MODEL), jnp.float32),
            pltpu.SemaphoreType.DMA,
            pltpu.SemaphoreType.DMA((LOG2_N,)),
            pltpu.SemaphoreType.DMA((LOG2_N,)),
        ],
        compiler_params=pltpu.CompilerParams(collective_id=0),
    )(x, Wq, Wo, k_loc, v_loc)


# device time: 66200 ns/iter; 2.4338x vs baseline; 1.0345x over previous
import jax
import jax.numpy as jnp
from jax import lax
from jax.experimental import pallas as pl
from jax.experimental.pallas import tpu as pltpu

N_DEV = 32
LOG2_N = 5
B = 2
SQ = 256
D_MODEL = 768
H_LOC = 8
GQA_GROUP = 4
KV_LOC = H_LOC // GQA_GROUP
DH = 64
SKV = 512
ROWS = B * SQ
N_CHUNKS = N_DEV
CHUNK_ROWS = ROWS // N_CHUNKS
CHUNKS_PER_B = SQ // CHUNK_ROWS


def _body(x_ref, wq_ref, wo_ref, k_ref, v_ref, out_ref,
          acc_ref, recv_ref, send_sem, rs_sems, ag_sems):
    my = lax.axis_index("i")

    barrier = pltpu.get_barrier_semaphore()
    for k in range(LOG2_N):
        pl.semaphore_signal(barrier, inc=1, device_id=(my ^ (1 << k),),
                            device_id_type=pl.DeviceIdType.MESH)
    pl.semaphore_wait(barrier, LOG2_N)

    def compute_partial(b):
        qb = jnp.dot(x_ref[b], wq_ref[...],
                     preferred_element_type=jnp.float32)
        outs = []
        for h in range(H_LOC):
            q = qb[:, h * DH:(h + 1) * DH]
            k = k_ref[b, h // GQA_GROUP]
            v = v_ref[b, h // GQA_GROUP]
            s = jnp.dot(q, k.T, preferred_element_type=jnp.float32) * 0.125
            m = jnp.max(s, axis=1, keepdims=True)
            p = jnp.exp(s - m)
            l = jnp.sum(p, axis=1, keepdims=True)
            outs.append(jnp.dot(p, v, preferred_element_type=jnp.float32) / l)
        o = jnp.concatenate(outs, axis=1)
        pb = jnp.dot(o, wo_ref[...],
                     preferred_element_type=jnp.float32)
        acc_ref[pl.ds(b * CHUNKS_PER_B, CHUNKS_PER_B)] = pb.reshape(
            CHUNKS_PER_B, CHUNK_ROWS, D_MODEL)

    bit0 = my & 1

    @pl.when(bit0 == 0)
    def _():
        compute_partial(1)

    @pl.when(bit0 == 1)
    def _():
        compute_partial(0)

    send_start0 = (1 - bit0) * 16
    send0 = pltpu.make_async_remote_copy(
        src_ref=acc_ref.at[pl.ds(send_start0, 16)],
        dst_ref=recv_ref.at[pl.ds(send_start0, 16)],
        send_sem=send_sem,
        recv_sem=rs_sems.at[0],
        device_id=(my ^ 1,),
        device_id_type=pl.DeviceIdType.MESH,
    )
    send0.start()

    @pl.when(bit0 == 0)
    def _():
        compute_partial(0)

    @pl.when(bit0 == 1)
    def _():
        compute_partial(1)

    keep_start0 = bit0 * 16
    recv0 = pltpu.make_async_remote_copy(
        src_ref=acc_ref.at[pl.ds(keep_start0, 16)],
        dst_ref=recv_ref.at[pl.ds(keep_start0, 16)],
        send_sem=send_sem,
        recv_sem=rs_sems.at[0],
        device_id=(my ^ 1,),
        device_id_type=pl.DeviceIdType.MESH,
    )
    recv0.wait_recv()
    acc_ref[pl.ds(keep_start0, 16)] = (
        acc_ref[pl.ds(keep_start0, 16)] + recv_ref[pl.ds(keep_start0, 16)])
    send0.wait_send()

    cur_start = keep_start0
    for k in range(1, LOG2_N):
        half = 16 >> k
        bit = (my >> k) & 1
        keep_start = cur_start + bit * half
        send_start = cur_start + (1 - bit) * half
        partner = my ^ (1 << k)
        send = pltpu.make_async_remote_copy(
            src_ref=acc_ref.at[pl.ds(send_start, half)],
            dst_ref=recv_ref.at[pl.ds(send_start, half)],
            send_sem=send_sem,
            recv_sem=rs_sems.at[k],
            device_id=(partner,),
            device_id_type=pl.DeviceIdType.MESH,
        )
        send.start()
        recv = pltpu.make_async_remote_copy(
            src_ref=acc_ref.at[pl.ds(keep_start, half)],
            dst_ref=recv_ref.at[pl.ds(keep_start, half)],
            send_sem=send_sem,
            recv_sem=rs_sems.at[k],
            device_id=(partner,),
            device_id_type=pl.DeviceIdType.MESH,
        )
        recv.wait_recv()
        acc_ref[pl.ds(keep_start, half)] = (
            acc_ref[pl.ds(keep_start, half)]
            + recv_ref[pl.ds(keep_start, half)])
        send.wait_send()
        cur_start = keep_start

    own_start = jnp.int32(0)
    for k in range(LOG2_N):
        own_start = own_start + (((my >> k) & 1) << (LOG2_N - 1 - k))

    for j in range(LOG2_N):
        sz = 1 << j
        partner = my ^ (16 >> j)
        partner_start = own_start ^ sz
        send = pltpu.make_async_remote_copy(
            src_ref=acc_ref.at[pl.ds(own_start, sz)],
            dst_ref=acc_ref.at[pl.ds(own_start, sz)],
            send_sem=send_sem,
            recv_sem=ag_sems.at[j],
            device_id=(partner,),
            device_id_type=pl.DeviceIdType.MESH,
        )
        send.start()
        recv = pltpu.make_async_remote_copy(
            src_ref=acc_ref.at[pl.ds(partner_start, sz)],
            dst_ref=acc_ref.at[pl.ds(partner_start, sz)],
            send_sem=send_sem,
            recv_sem=ag_sems.at[j],
            device_id=(partner,),
            device_id_type=pl.DeviceIdType.MESH,
        )
        recv.wait_recv()
        send.wait_send()
        own_start = jnp.minimum(own_start, partner_start)

    for b in range(B):
        out_ref[b] = acc_ref[b * CHUNKS_PER_B:(b + 1) * CHUNKS_PER_B].reshape(
            SQ, D_MODEL)


def kernel(x, Wq, Wo, K_ext, V_ext):
    i = lax.axis_index("i")
    k_loc = lax.dynamic_slice_in_dim(K_ext, KV_LOC * i, KV_LOC, axis=2)
    v_loc = lax.dynamic_slice_in_dim(V_ext, KV_LOC * i, KV_LOC, axis=2)
    k_loc = k_loc.transpose(0, 2, 1, 3)
    v_loc = v_loc.transpose(0, 2, 1, 3)

    return pl.pallas_call(
        _body,
        out_shape=jax.ShapeDtypeStruct((B, SQ, D_MODEL), jnp.float32),
        in_specs=[pl.BlockSpec(memory_space=pltpu.VMEM)] * 5,
        out_specs=pl.BlockSpec(memory_space=pltpu.VMEM),
        scratch_shapes=[
            pltpu.VMEM((N_CHUNKS, CHUNK_ROWS, D_MODEL), jnp.float32),
            pltpu.VMEM((N_CHUNKS, CHUNK_ROWS, D_MODEL), jnp.float32),
            pltpu.SemaphoreType.DMA,
            pltpu.SemaphoreType.DMA((LOG2_N,)),
            pltpu.SemaphoreType.DMA((LOG2_N,)),
        ],
        compiler_params=pltpu.CompilerParams(collective_id=0),
    )(x, Wq, Wo, k_loc, v_loc)


# device time: 48122 ns/iter; 3.3482x vs baseline; 1.3757x over previous
import jax
import jax.numpy as jnp
from jax import lax
from jax.experimental import pallas as pl
from jax.experimental.pallas import tpu as pltpu

N_DEV = 32
LOG2_N = 5
B = 2
SQ = 256
D_MODEL = 768
H_LOC = 8
GQA_GROUP = 4
KV_LOC = H_LOC // GQA_GROUP
DH = 64
SKV = 512
ROWS = B * SQ
N_CHUNKS = N_DEV
CHUNK_ROWS = ROWS // N_CHUNKS
CHUNKS_PER_B = SQ // CHUNK_ROWS

RS_OFS = [0, 16, 24, 28, 30]
AG_OFS = [0, 1, 3, 7, 15]


def _body(x_ref, wq_ref, wo_ref, k_ref, v_ref, out_ref,
          acc_ref, rs_send_ref, rs_recv_ref, ag_send_ref, ag_recv_ref,
          send_sem, rs_sems, ag_sems):
    my = lax.axis_index("i")

    barrier = pltpu.get_barrier_semaphore()
    for k in range(LOG2_N):
        pl.semaphore_signal(barrier, inc=1, device_id=(my ^ (1 << k),),
                            device_id_type=pl.DeviceIdType.MESH)
    pl.semaphore_wait(barrier, LOG2_N)

    def compute_partial(b):
        qb = jnp.dot(x_ref[b], wq_ref[...],
                     preferred_element_type=jnp.float32)
        outs = []
        for h in range(H_LOC):
            q = qb[:, h * DH:(h + 1) * DH]
            k = k_ref[b, h // GQA_GROUP]
            v = v_ref[b, h // GQA_GROUP]
            s = jnp.dot(q, k.T, preferred_element_type=jnp.float32) * 0.125
            m = jnp.max(s, axis=1, keepdims=True)
            p = jnp.exp(s - m)
            l = jnp.sum(p, axis=1, keepdims=True)
            outs.append(jnp.dot(p, v, preferred_element_type=jnp.float32) / l)
        o = jnp.concatenate(outs, axis=1)
        pb = jnp.dot(o, wo_ref[...],
                     preferred_element_type=jnp.float32)
        acc_ref[pl.ds(b * CHUNKS_PER_B, CHUNKS_PER_B)] = pb.reshape(
            CHUNKS_PER_B, CHUNK_ROWS, D_MODEL)

    bit0 = my & 1
    send_start0 = (1 - bit0) * 16
    keep_start0 = bit0 * 16

    @pl.when(bit0 == 0)
    def _():
        compute_partial(1)

    @pl.when(bit0 == 1)
    def _():
        compute_partial(0)

    rs_send_ref[pl.ds(RS_OFS[0], 16)] = acc_ref[
        pl.ds(send_start0, 16)].astype(jnp.bfloat16)
    send0 = pltpu.make_async_remote_copy(
        src_ref=rs_send_ref.at[pl.ds(RS_OFS[0], 16)],
        dst_ref=rs_recv_ref.at[pl.ds(RS_OFS[0], 16)],
        send_sem=send_sem,
        recv_sem=rs_sems.at[0],
        device_id=(my ^ 1,),
        device_id_type=pl.DeviceIdType.MESH,
    )
    send0.start()

    @pl.when(bit0 == 0)
    def _():
        compute_partial(0)

    @pl.when(bit0 == 1)
    def _():
        compute_partial(1)

    recv0 = pltpu.make_async_remote_copy(
        src_ref=rs_send_ref.at[pl.ds(RS_OFS[0], 16)],
        dst_ref=rs_recv_ref.at[pl.ds(RS_OFS[0], 16)],
        send_sem=send_sem,
        recv_sem=rs_sems.at[0],
        device_id=(my ^ 1,),
        device_id_type=pl.DeviceIdType.MESH,
    )
    recv0.wait_recv()
    acc_ref[pl.ds(keep_start0, 16)] = (
        acc_ref[pl.ds(keep_start0, 16)]
        + rs_recv_ref[pl.ds(RS_OFS[0], 16)].astype(jnp.float32))
    send0.wait_send()

    cur_start = keep_start0
    for k in range(1, LOG2_N):
        half = 16 >> k
        ofs = RS_OFS[k]
        bit = (my >> k) & 1
        keep_start = cur_start + bit * half
        send_start = cur_start + (1 - bit) * half
        partner = my ^ (1 << k)
        rs_send_ref[pl.ds(ofs, half)] = acc_ref[
            pl.ds(send_start, half)].astype(jnp.bfloat16)
        send = pltpu.make_async_remote_copy(
            src_ref=rs_send_ref.at[pl.ds(ofs, half)],
            dst_ref=rs_recv_ref.at[pl.ds(ofs, half)],
            send_sem=send_sem,
            recv_sem=rs_sems.at[k],
            device_id=(partner,),
            device_id_type=pl.DeviceIdType.MESH,
        )
        send.start()
        recv = pltpu.make_async_remote_copy(
            src_ref=rs_send_ref.at[pl.ds(ofs, half)],
            dst_ref=rs_recv_ref.at[pl.ds(ofs, half)],
            send_sem=send_sem,
            recv_sem=rs_sems.at[k],
            device_id=(partner,),
            device_id_type=pl.DeviceIdType.MESH,
        )
        recv.wait_recv()
        acc_ref[pl.ds(keep_start, half)] = (
            acc_ref[pl.ds(keep_start, half)]
            + rs_recv_ref[pl.ds(ofs, half)].astype(jnp.float32))
        send.wait_send()
        cur_start = keep_start

    own_start = jnp.int32(0)
    for k in range(LOG2_N):
        own_start = own_start + (((my >> k) & 1) << (LOG2_N - 1 - k))

    for j in range(LOG2_N):
        sz = 1 << j
        ofs = AG_OFS[j]
        partner = my ^ (16 >> j)
        partner_start = own_start ^ sz
        ag_send_ref[pl.ds(ofs, sz)] = acc_ref[
            pl.ds(own_start, sz)].astype(jnp.bfloat16)
        send = pltpu.make_async_remote_copy(
            src_ref=ag_send_ref.at[pl.ds(ofs, sz)],
            dst_ref=ag_recv_ref.at[pl.ds(ofs, sz)],
            send_sem=send_sem,
            recv_sem=ag_sems.at[j],
            device_id=(partner,),
            device_id_type=pl.DeviceIdType.MESH,
        )
        send.start()
        recv = pltpu.make_async_remote_copy(
            src_ref=ag_send_ref.at[pl.ds(ofs, sz)],
            dst_ref=ag_recv_ref.at[pl.ds(ofs, sz)],
            send_sem=send_sem,
            recv_sem=ag_sems.at[j],
            device_id=(partner,),
            device_id_type=pl.DeviceIdType.MESH,
        )
        if j < LOG2_N - 1:
            recv.wait_recv()
            acc_ref[pl.ds(partner_start, sz)] = ag_recv_ref[
                pl.ds(ofs, sz)].astype(jnp.float32)
            send.wait_send()
            own_start = jnp.minimum(own_start, partner_start)
        else:
            bb = own_start // CHUNKS_PER_B
            out_ref[pl.ds(bb, 1)] = acc_ref[pl.ds(own_start, 16)].reshape(
                1, SQ, D_MODEL)
            recv.wait_recv()
            out_ref[pl.ds(1 - bb, 1)] = ag_recv_ref[
                pl.ds(ofs, sz)].astype(jnp.float32).reshape(1, SQ, D_MODEL)
            send.wait_send()


def kernel(x, Wq, Wo, K_ext, V_ext):
    i = lax.axis_index("i")
    k_loc = lax.dynamic_slice_in_dim(K_ext, KV_LOC * i, KV_LOC, axis=2)
    v_loc = lax.dynamic_slice_in_dim(V_ext, KV_LOC * i, KV_LOC, axis=2)
    k_loc = k_loc.transpose(0, 2, 1, 3)
    v_loc = v_loc.transpose(0, 2, 1, 3)

    return pl.pallas_call(
        _body,
        out_shape=jax.ShapeDtypeStruct((B, SQ, D_MODEL), jnp.float32),
        in_specs=[pl.BlockSpec(memory_space=pltpu.VMEM)] * 5,
        out_specs=pl.BlockSpec(memory_space=pltpu.VMEM),
        scratch_shapes=[
            pltpu.VMEM((N_CHUNKS, CHUNK_ROWS, D_MODEL), jnp.float32),
            pltpu.VMEM((31, CHUNK_ROWS, D_MODEL), jnp.bfloat16),
            pltpu.VMEM((31, CHUNK_ROWS, D_MODEL), jnp.bfloat16),
            pltpu.VMEM((31, CHUNK_ROWS, D_MODEL), jnp.bfloat16),
            pltpu.VMEM((31, CHUNK_ROWS, D_MODEL), jnp.bfloat16),
            pltpu.SemaphoreType.DMA,
            pltpu.SemaphoreType.DMA((LOG2_N,)),
            pltpu.SemaphoreType.DMA((LOG2_N,)),
        ],
        compiler_params=pltpu.CompilerParams(collective_id=0),
    )(x, Wq, Wo, k_loc, v_loc)


# device time: 45961 ns/iter; 3.5056x vs baseline; 1.0470x over previous
import jax
import jax.numpy as jnp
from jax import lax
from jax.experimental import pallas as pl
from jax.experimental.pallas import tpu as pltpu

N_DEV = 32
LOG2_N = 5
B = 2
SQ = 256
D_MODEL = 768
H_LOC = 8
GQA_GROUP = 4
KV_LOC = H_LOC // GQA_GROUP
DH = 64
SKV = 512
ROWS = B * SQ
N_CHUNKS = N_DEV
CHUNK_ROWS = ROWS // N_CHUNKS
CHUNKS_PER_B = SQ // CHUNK_ROWS

RS_MASKS = [1, 2, 8, 4]
RS_BITS = [0, 1, 3, 2]
RS_HALVES = [16, 8, 4, 2]
RS_OFS = [0, 16, 24, 28]
ARE_OFS = 30
AG_OFS = [0, 2, 6, 14]


def _body(x_ref, wq_ref, wo_ref, k_ref, v_ref, out_ref,
          acc_ref, rs_send_ref, rs_recv_ref, ag_send_ref, ag_recv_ref,
          send_sem, rs_sems, ag_sems):
    my = lax.axis_index("i")

    barrier = pltpu.get_barrier_semaphore()
    for k in range(LOG2_N):
        pl.semaphore_signal(barrier, inc=1, device_id=(my ^ (1 << k),),
                            device_id_type=pl.DeviceIdType.MESH)
    pl.semaphore_wait(barrier, LOG2_N)

    def compute_partial(b):
        qb = jnp.dot(x_ref[b], wq_ref[...],
                     preferred_element_type=jnp.float32)
        outs = []
        for h in range(H_LOC):
            q = qb[:, h * DH:(h + 1) * DH]
            k = k_ref[b, h // GQA_GROUP]
            v = v_ref[b, h // GQA_GROUP]
            s = jnp.dot(q, k.T, preferred_element_type=jnp.float32) * 0.125
            m = jnp.max(s, axis=1, keepdims=True)
            p = jnp.exp(s - m)
            l = jnp.sum(p, axis=1, keepdims=True)
            outs.append(jnp.dot(p, v, preferred_element_type=jnp.float32) / l)
        o = jnp.concatenate(outs, axis=1)
        pb = jnp.dot(o, wo_ref[...],
                     preferred_element_type=jnp.float32)
        acc_ref[pl.ds(b * CHUNKS_PER_B, CHUNKS_PER_B)] = pb.reshape(
            CHUNKS_PER_B, CHUNK_ROWS, D_MODEL)

    def rs_exchange(k, send_start, keep_start, half, partner):
        ofs = RS_OFS[k]
        rs_send_ref[pl.ds(ofs, half)] = acc_ref[
            pl.ds(send_start, half)].astype(jnp.bfloat16)
        send = pltpu.make_async_remote_copy(
            src_ref=rs_send_ref.at[pl.ds(ofs, half)],
            dst_ref=rs_recv_ref.at[pl.ds(ofs, half)],
            send_sem=send_sem,
            recv_sem=rs_sems.at[k],
            device_id=(partner,),
            device_id_type=pl.DeviceIdType.MESH,
        )
        send.start()
        send.wait_recv()
        acc_ref[pl.ds(keep_start, half)] = (
            acc_ref[pl.ds(keep_start, half)]
            + rs_recv_ref[pl.ds(ofs, half)].astype(jnp.float32))
        send.wait_send()

    bit0 = my & 1
    send_start0 = (1 - bit0) * 16
    keep_start0 = bit0 * 16

    @pl.when(bit0 == 0)
    def _():
        compute_partial(1)

    @pl.when(bit0 == 1)
    def _():
        compute_partial(0)

    rs_send_ref[pl.ds(RS_OFS[0], 16)] = acc_ref[
        pl.ds(send_start0, 16)].astype(jnp.bfloat16)
    send0 = pltpu.make_async_remote_copy(
        src_ref=rs_send_ref.at[pl.ds(RS_OFS[0], 16)],
        dst_ref=rs_recv_ref.at[pl.ds(RS_OFS[0], 16)],
        send_sem=send_sem,
        recv_sem=rs_sems.at[0],
        device_id=(my ^ 1,),
        device_id_type=pl.DeviceIdType.MESH,
    )
    send0.start()

    @pl.when(bit0 == 0)
    def _():
        compute_partial(0)

    @pl.when(bit0 == 1)
    def _():
        compute_partial(1)

    send0.wait_recv()
    acc_ref[pl.ds(keep_start0, 16)] = (
        acc_ref[pl.ds(keep_start0, 16)]
        + rs_recv_ref[pl.ds(RS_OFS[0], 16)].astype(jnp.float32))
    send0.wait_send()

    cur_start = keep_start0
    starts = [jnp.int32(0), cur_start]
    bits = [bit0]
    for k in range(1, 4):
        half = RS_HALVES[k]
        bit = (my >> RS_BITS[k]) & 1
        keep_start = cur_start + bit * half
        send_start = cur_start + (1 - bit) * half
        rs_exchange(k, send_start, keep_start, half, my ^ RS_MASKS[k])
        cur_start = keep_start
        starts.append(cur_start)
        bits.append(bit)

    rs_send_ref[pl.ds(ARE_OFS, 2)] = acc_ref[
        pl.ds(cur_start, 2)].astype(jnp.bfloat16)
    are = pltpu.make_async_remote_copy(
        src_ref=rs_send_ref.at[pl.ds(ARE_OFS, 2)],
        dst_ref=rs_recv_ref.at[pl.ds(ARE_OFS, 2)],
        send_sem=send_sem,
        recv_sem=rs_sems.at[4],
        device_id=(my ^ 16,),
        device_id_type=pl.DeviceIdType.MESH,
    )
    are.start()
    are.wait_recv()
    acc_ref[pl.ds(cur_start, 2)] = (
        acc_ref[pl.ds(cur_start, 2)]
        + rs_recv_ref[pl.ds(ARE_OFS, 2)].astype(jnp.float32))
    are.wait_send()

    for j in range(4):
        kk = 3 - j
        sz = RS_HALVES[kk]
        ofs = AG_OFS[j]
        partner = my ^ RS_MASKS[kk]
        own_start = starts[kk + 1]
        partner_start = starts[kk] + (1 - bits[kk]) * sz
        ag_send_ref[pl.ds(ofs, sz)] = acc_ref[
            pl.ds(own_start, sz)].astype(jnp.bfloat16)
        send = pltpu.make_async_remote_copy(
            src_ref=ag_send_ref.at[pl.ds(ofs, sz)],
            dst_ref=ag_recv_ref.at[pl.ds(ofs, sz)],
            send_sem=send_sem,
            recv_sem=ag_sems.at[j],
            device_id=(partner,),
            device_id_type=pl.DeviceIdType.MESH,
        )
        send.start()
        if j < 3:
            send.wait_recv()
            acc_ref[pl.ds(partner_start, sz)] = ag_recv_ref[
                pl.ds(ofs, sz)].astype(jnp.float32)
            send.wait_send()
        else:
            bb = own_start // CHUNKS_PER_B
            out_ref[pl.ds(bb, 1)] = acc_ref[pl.ds(own_start, 16)].reshape(
                1, SQ, D_MODEL)
            send.wait_recv()
            out_ref[pl.ds(1 - bb, 1)] = ag_recv_ref[
                pl.ds(ofs, sz)].astype(jnp.float32).reshape(1, SQ, D_MODEL)
            send.wait_send()


def kernel(x, Wq, Wo, K_ext, V_ext):
    i = lax.axis_index("i")
    k_loc = lax.dynamic_slice_in_dim(K_ext, KV_LOC * i, KV_LOC, axis=2)
    v_loc = lax.dynamic_slice_in_dim(V_ext, KV_LOC * i, KV_LOC, axis=2)
    k_loc = k_loc.transpose(0, 2, 1, 3)
    v_loc = v_loc.transpose(0, 2, 1, 3)

    return pl.pallas_call(
        _body,
        out_shape=jax.ShapeDtypeStruct((B, SQ, D_MODEL), jnp.float32),
        in_specs=[pl.BlockSpec(memory_space=pltpu.VMEM)] * 5,
        out_specs=pl.BlockSpec(memory_space=pltpu.VMEM),
        scratch_shapes=[
            pltpu.VMEM((N_CHUNKS, CHUNK_ROWS, D_MODEL), jnp.float32),
            pltpu.VMEM((32, CHUNK_ROWS, D_MODEL), jnp.bfloat16),
            pltpu.VMEM((32, CHUNK_ROWS, D_MODEL), jnp.bfloat16),
            pltpu.VMEM((32, CHUNK_ROWS, D_MODEL), jnp.bfloat16),
            pltpu.VMEM((32, CHUNK_ROWS, D_MODEL), jnp.bfloat16),
            pltpu.SemaphoreType.DMA,
            pltpu.SemaphoreType.DMA((LOG2_N,)),
            pltpu.SemaphoreType.DMA((4,)),
        ],
        compiler_params=pltpu.CompilerParams(collective_id=0),
    )(x, Wq, Wo, k_loc, v_loc)


# device time: 37631 ns/iter; 4.2816x vs baseline; 1.2214x over previous
import jax
import jax.numpy as jnp
from jax import lax
from jax.experimental import pallas as pl
from jax.experimental.pallas import tpu as pltpu

N_DEV = 32
B = 2
SQ = 256
D_MODEL = 768
H_LOC = 8
GQA_GROUP = 4
KV_LOC = H_LOC // GQA_GROUP
DH = 64
SKV = 512
ROWS = B * SQ
N_CHUNKS = N_DEV
CHUNK_ROWS = ROWS // N_CHUNKS
CHUNKS_PER_B = SQ // CHUNK_ROWS

BARRIER_MASKS = (1, 2, 3, 4, 8, 12, 16)


def _body(x_ref, wq_ref, wo_ref, k_ref, v_ref, out_ref,
          acc_ref, rs_send_ref, rs_recv_ref, ag_send_ref, ag_recv_ref,
          send_sems, rs_sems, ag_sems):
    my = lax.axis_index("i")
    qa = my & 3
    qb = (my >> 2) & 3
    b_keep = (my >> 1) & 1

    barrier = pltpu.get_barrier_semaphore()
    for msk in BARRIER_MASKS:
        pl.semaphore_signal(barrier, inc=1, device_id=(my ^ msk,),
                            device_id_type=pl.DeviceIdType.MESH)
    pl.semaphore_wait(barrier, len(BARRIER_MASKS))

    def compute_partial(b):
        qmat = jnp.dot(x_ref[b], wq_ref[...],
                       preferred_element_type=jnp.float32)
        outs = []
        for h in range(H_LOC):
            q = qmat[:, h * DH:(h + 1) * DH]
            k = k_ref[b, h // GQA_GROUP]
            v = v_ref[b, h // GQA_GROUP]
            s = jnp.dot(q, k.T, preferred_element_type=jnp.float32) * 0.125
            m = jnp.max(s, axis=1, keepdims=True)
            p = jnp.exp(s - m)
            l = jnp.sum(p, axis=1, keepdims=True)
            outs.append(jnp.dot(p, v, preferred_element_type=jnp.float32) / l)
        o = jnp.concatenate(outs, axis=1)
        pb = jnp.dot(o, wo_ref[...],
                     preferred_element_type=jnp.float32)
        acc_ref[pl.ds(b * CHUNKS_PER_B, CHUNKS_PER_B)] = pb.reshape(
            CHUNKS_PER_B, CHUNK_ROWS, D_MODEL)

    def exchange(send_buf, send_ofs, recv_buf, recv_ofs, n, sem_i, recv_sem,
                 partner):
        rdma = pltpu.make_async_remote_copy(
            src_ref=send_buf.at[pl.ds(send_ofs, n)],
            dst_ref=recv_buf.at[pl.ds(recv_ofs, n)],
            send_sem=send_sems.at[sem_i],
            recv_sem=recv_sem,
            device_id=(partner,),
            device_id_type=pl.DeviceIdType.MESH,
        )
        rdma.start()
        return rdma

    @pl.when(b_keep == 0)
    def _():
        compute_partial(1)

    @pl.when(b_keep == 1)
    def _():
        compute_partial(0)

    rdma_a = {}
    for d in (2, 3):
        ofs = (d - 1) * 8
        rs_send_ref[pl.ds(ofs, 8)] = acc_ref[
            pl.ds((qa ^ d) * 8, 8)].astype(jnp.bfloat16)
        rdma_a[d] = exchange(rs_send_ref, ofs, rs_recv_ref, ofs, 8,
                             d - 1, rs_sems.at[d - 1], my ^ d)

    @pl.when(b_keep == 0)
    def _():
        compute_partial(0)

    @pl.when(b_keep == 1)
    def _():
        compute_partial(1)

    rs_send_ref[pl.ds(0, 8)] = acc_ref[pl.ds((qa ^ 1) * 8, 8)].astype(
        jnp.bfloat16)
    rdma_a[1] = exchange(rs_send_ref, 0, rs_recv_ref, 0, 8,
                         0, rs_sems.at[0], my ^ 1)

    for d in (1, 3, 2):
        rdma_a[d].wait_recv()
    acc_ref[pl.ds(qa * 8, 8)] = (
        acc_ref[pl.ds(qa * 8, 8)]
        + rs_recv_ref[pl.ds(0, 8)].astype(jnp.float32)
        + rs_recv_ref[pl.ds(8, 8)].astype(jnp.float32)
        + rs_recv_ref[pl.ds(16, 8)].astype(jnp.float32))
    for d in (1, 2, 3):
        rdma_a[d].wait_send()

    rb = qa * 8
    rdma_b = {}
    for d in (1, 2, 3):
        ofs = 24 + (d - 1) * 2
        rs_send_ref[pl.ds(ofs, 2)] = acc_ref[
            pl.ds(rb + (qb ^ d) * 2, 2)].astype(jnp.bfloat16)
        rdma_b[d] = exchange(rs_send_ref, ofs, rs_recv_ref, ofs, 2,
                             d - 1, rs_sems.at[3 + d - 1], my ^ (d << 2))
    for d in (1, 2, 3):
        rdma_b[d].wait_recv()
    fs = rb + qb * 2
    acc_ref[pl.ds(fs, 2)] = (
        acc_ref[pl.ds(fs, 2)]
        + rs_recv_ref[pl.ds(24, 2)].astype(jnp.float32)
        + rs_recv_ref[pl.ds(26, 2)].astype(jnp.float32)
        + rs_recv_ref[pl.ds(28, 2)].astype(jnp.float32))
    for d in (1, 2, 3):
        rdma_b[d].wait_send()

    rs_send_ref[pl.ds(30, 2)] = acc_ref[pl.ds(fs, 2)].astype(jnp.bfloat16)
    are = exchange(rs_send_ref, 30, rs_recv_ref, 30, 2,
                   0, rs_sems.at[6], my ^ 16)
    are.wait_recv()
    acc_ref[pl.ds(fs, 2)] = (
        acc_ref[pl.ds(fs, 2)]
        + rs_recv_ref[pl.ds(30, 2)].astype(jnp.float32))
    are.wait_send()

    ag_send_ref[pl.ds(0, 2)] = acc_ref[pl.ds(fs, 2)].astype(jnp.bfloat16)
    rdma_gb = {}
    for d in (1, 2, 3):
        rdma_gb[d] = exchange(ag_send_ref, 0, ag_recv_ref, (d - 1) * 2, 2,
                              d - 1, ag_sems.at[d - 1], my ^ (d << 2))
    for d in (1, 2, 3):
        rdma_gb[d].wait_recv()
        acc_ref[pl.ds(rb + (qb ^ d) * 2, 2)] = ag_recv_ref[
            pl.ds((d - 1) * 2, 2)].astype(jnp.float32)
    for d in (1, 2, 3):
        rdma_gb[d].wait_send()

    ag_send_ref[pl.ds(2, 8)] = acc_ref[pl.ds(rb, 8)].astype(jnp.bfloat16)
    rdma_ga = {}
    for d in (1, 2, 3):
        rdma_ga[d] = exchange(ag_send_ref, 2, ag_recv_ref, 6 + (d - 1) * 8, 8,
                              d - 1, ag_sems.at[3 + d - 1], my ^ d)
    for d in (1, 2, 3):
        rdma_ga[d].wait_recv()
        acc_ref[pl.ds((qa ^ d) * 8, 8)] = ag_recv_ref[
            pl.ds(6 + (d - 1) * 8, 8)].astype(jnp.float32)
    for d in (1, 2, 3):
        rdma_ga[d].wait_send()

    for b in range(B):
        out_ref[b] = acc_ref[b * CHUNKS_PER_B:(b + 1) * CHUNKS_PER_B].reshape(
            SQ, D_MODEL)


def kernel(x, Wq, Wo, K_ext, V_ext):
    i = lax.axis_index("i")
    k_loc = lax.dynamic_slice_in_dim(K_ext, KV_LOC * i, KV_LOC, axis=2)
    v_loc = lax.dynamic_slice_in_dim(V_ext, KV_LOC * i, KV_LOC, axis=2)
    k_loc = k_loc.transpose(0, 2, 1, 3)
    v_loc = v_loc.transpose(0, 2, 1, 3)

    return pl.pallas_call(
        _body,
        out_shape=jax.ShapeDtypeStruct((B, SQ, D_MODEL), jnp.float32),
        in_specs=[pl.BlockSpec(memory_space=pltpu.VMEM)] * 5,
        out_specs=pl.BlockSpec(memory_space=pltpu.VMEM),
        scratch_shapes=[
            pltpu.VMEM((N_CHUNKS, CHUNK_ROWS, D_MODEL), jnp.float32),
            pltpu.VMEM((32, CHUNK_ROWS, D_MODEL), jnp.bfloat16),
            pltpu.VMEM((32, CHUNK_ROWS, D_MODEL), jnp.bfloat16),
            pltpu.VMEM((10, CHUNK_ROWS, D_MODEL), jnp.bfloat16),
            pltpu.VMEM((30, CHUNK_ROWS, D_MODEL), jnp.bfloat16),
            pltpu.SemaphoreType.DMA((3,)),
            pltpu.SemaphoreType.DMA((7,)),
            pltpu.SemaphoreType.DMA((6,)),
        ],
        compiler_params=pltpu.CompilerParams(collective_id=0),
    )(x, Wq, Wo, k_loc, v_loc)
